# Initial kernel scaffold; baseline (speedup 1.0000x reference)
import jax
import jax.numpy as jnp
from jax.experimental import pallas as pl


def kernel(x, edge_index, W, b):
    raise NotImplementedError("write your pallas kernel here")


# speedup vs baseline: 15.3835x; 15.3835x over previous
import functools

import jax
import jax.numpy as jnp
from jax import lax
from jax.experimental import pallas as pl
from jax.experimental.pallas import tpu as pltpu
from jax.experimental.pallas import tpu_sc as plsc

NC = 2
NS = 16
NW = NC * NS
K = 80
DEGW = 128


def _sc_mesh():
    return plsc.VectorSubcoreMesh(
        core_axis_name="c", subcore_axis_name="s", num_cores=NC, num_subcores=NS
    )


def _stripe(n, s):
    sz = ((n + NS - 1) // NS + 7) // 8 * 8
    base = jnp.minimum(s * sz, n - sz)
    return base, sz


def _make_deg_call(n, e):
    ew = e // NW

    @functools.partial(
        pl.kernel,
        out_type=jax.ShapeDtypeStruct((NC, n, DEGW), jnp.float32),
        mesh=_sc_mesh(),
        scratch_types=[
            pltpu.VMEM((K,), jnp.int32),
            pltpu.VMEM((K, DEGW), jnp.float32),
            pltpu.VMEM_SHARED((n, DEGW), jnp.float32),
        ],
    )
    def deg_kernel(row_hbm, ones_hbm, zeros_hbm, out_hbm, idx_v, ones_v, acc_sh):
        c = lax.axis_index("c")
        s = lax.axis_index("s")
        wid = s * NC + c
        sb, sz = _stripe(n, s)
        pltpu.sync_copy(zeros_hbm.at[pl.ds(sb, sz)], acc_sh.at[pl.ds(sb, sz)])
        pltpu.sync_copy(ones_hbm, ones_v)
        plsc.subcore_barrier()

        def body(j, carry):
            base = wid * ew + j * K
            pltpu.sync_copy(row_hbm.at[pl.ds(base, K)], idx_v)
            pltpu.sync_copy(ones_v, acc_sh.at[idx_v], add=True)
            return carry

        lax.fori_loop(0, ew // K, body, 0)
        plsc.subcore_barrier()
        pltpu.sync_copy(acc_sh.at[pl.ds(sb, sz)], out_hbm.at[c, pl.ds(sb, sz)])

    return deg_kernel


def _make_scatter_call(n, d, e):
    ew = e // NW

    @functools.partial(
        pl.kernel,
        out_type=jax.ShapeDtypeStruct((NC, n, d), jnp.float32),
        mesh=_sc_mesh(),
        scratch_types=[
            pltpu.VMEM((K,), jnp.int32),
            pltpu.VMEM((K,), jnp.int32),
            pltpu.VMEM((K, d), jnp.float32),
            pltpu.VMEM_SHARED((n, d), jnp.float32),
            pltpu.SemaphoreType.DMA,
        ],
    )
    def scatter_kernel(h2_hbm, row_hbm, col_hbm, zeros_hbm, out_hbm,
                       colv, rowv, rows_v, acc_sh, sem):
        c = lax.axis_index("c")
        s = lax.axis_index("s")
        wid = s * NC + c
        sb, sz = _stripe(n, s)
        pltpu.sync_copy(zeros_hbm.at[pl.ds(sb, sz)], acc_sh.at[pl.ds(sb, sz)])
        plsc.subcore_barrier()

        def body(j, carry):
            base = wid * ew + j * K
            pltpu.sync_copy(col_hbm.at[pl.ds(base, K)], colv)
            pltpu.sync_copy(row_hbm.at[pl.ds(base, K)], rowv)
            pltpu.async_copy(h2_hbm.at[colv], rows_v, sem).wait()
            pltpu.sync_copy(rows_v, acc_sh.at[rowv], add=True)
            return carry

        lax.fori_loop(0, ew // K, body, 0)
        plsc.subcore_barrier()
        pltpu.sync_copy(acc_sh.at[pl.ds(sb, sz)], out_hbm.at[c, pl.ds(sb, sz)])

    return scatter_kernel


def _h2_body(x_ref, wt_ref, b_ref, degp_ref, h2_ref):
    parts = degp_ref[...]
    deg = parts[0] + parts[1]
    dinv = lax.rsqrt(deg[:, 0:1] + 1.0)
    h = jnp.dot(x_ref[...], wt_ref[...], preferred_element_type=jnp.float32)
    h2_ref[...] = (h + b_ref[...]) * dinv


def _final_body(parts_ref, h2_ref, degp_ref, out_ref):
    dparts = degp_ref[...]
    deg = dparts[0] + dparts[1]
    dinv = lax.rsqrt(deg[:, 0:1] + 1.0)
    acc = parts_ref[0] + parts_ref[1] + h2_ref[...]
    out_ref[...] = acc * dinv


def kernel(x, edge_index, W, b):
    n, d_in = x.shape
    d_out = W.shape[0]
    e = edge_index.shape[1]
    assert e % (NW * K) == 0 and n % NS == 0

    row = edge_index[0].astype(jnp.int32)
    col = edge_index[1].astype(jnp.int32)
    w_t = W.T
    b2 = b.reshape(1, d_out)
    ones_k = jnp.ones((K, DEGW), jnp.float32)
    zeros_nd = jnp.zeros((n, d_out), jnp.float32)

    degp = _make_deg_call(n, e)(row, ones_k, zeros_nd)

    blk = 1000
    grid = (n // blk,)
    h2 = pl.pallas_call(
        _h2_body,
        grid=grid,
        in_specs=[
            pl.BlockSpec((blk, d_in), lambda i: (i, 0)),
            pl.BlockSpec((d_in, d_out), lambda i: (0, 0)),
            pl.BlockSpec((1, d_out), lambda i: (0, 0)),
            pl.BlockSpec((NC, blk, DEGW), lambda i: (0, i, 0)),
        ],
        out_specs=pl.BlockSpec((blk, d_out), lambda i: (i, 0)),
        out_shape=jax.ShapeDtypeStruct((n, d_out), jnp.float32),
    )(x, w_t, b2, degp)

    parts = _make_scatter_call(n, d_out, e)(h2, row, col, zeros_nd)

    out = pl.pallas_call(
        _final_body,
        grid=grid,
        in_specs=[
            pl.BlockSpec((NC, blk, d_out), lambda i: (0, i, 0)),
            pl.BlockSpec((blk, d_out), lambda i: (i, 0)),
            pl.BlockSpec((NC, blk, DEGW), lambda i: (0, i, 0)),
        ],
        out_specs=pl.BlockSpec((blk, d_out), lambda i: (i, 0)),
        out_shape=jax.ShapeDtypeStruct((n, d_out), jnp.float32),
    )(parts, h2, degp)
    return out


# speedup vs baseline: 31.0065x; 2.0156x over previous
import functools

import jax
import jax.numpy as jnp
from jax import lax
from jax.experimental import pallas as pl
from jax.experimental.pallas import tpu as pltpu
from jax.experimental.pallas import tpu_sc as plsc

NC = 2
NS = 16
NW = NC * NS
K = 100
CH = 100
FB = 4
DEGW = 128


def _sc_mesh():
    return plsc.VectorSubcoreMesh(
        core_axis_name="c", subcore_axis_name="s", num_cores=NC, num_subcores=NS
    )


def _stripe(n, s):
    sz = ((n + NS - 1) // NS + 7) // 8 * 8
    base = jnp.minimum(s * sz, n - sz)
    return base, sz


def _make_deg_call(n, e):
    @functools.partial(
        pl.kernel,
        out_type=jax.ShapeDtypeStruct((NC, n, DEGW), jnp.float32),
        mesh=_sc_mesh(),
        scratch_types=[
            pltpu.VMEM((CH, K), jnp.int32),
            pltpu.VMEM((K, DEGW), jnp.float32),
            pltpu.VMEM_SHARED((n, DEGW), jnp.float32),
            pltpu.SemaphoreType.DMA,
        ],
    )
    def deg_kernel(row_hbm, ones_hbm, zeros_hbm, out_hbm, rowv, ones_v,
                   acc_sh, sem):
        c = lax.axis_index("c")
        s = lax.axis_index("s")
        wid = s * NC + c
        sb, sz = _stripe(n, s)
        pltpu.sync_copy(zeros_hbm.at[pl.ds(sb, sz)], acc_sh.at[pl.ds(sb, sz)])
        pltpu.sync_copy(ones_hbm, ones_v)
        pltpu.sync_copy(row_hbm.at[wid], rowv)
        plsc.subcore_barrier()

        def body(g, carry):
            j = g * FB
            for t in range(FB):
                pltpu.async_copy(ones_v, acc_sh.at[rowv.at[j + t]], sem,
                                 add=True)
            for t in range(FB):
                pltpu.make_async_copy(ones_v, acc_sh.at[rowv.at[j + t]],
                                      sem).wait()
            return carry

        lax.fori_loop(0, CH // FB, body, 0)
        plsc.subcore_barrier()
        pltpu.sync_copy(acc_sh.at[pl.ds(sb, sz)], out_hbm.at[c, pl.ds(sb, sz)])

    return deg_kernel


def _make_scatter_call(n, d, e):
    @functools.partial(
        pl.kernel,
        out_type=jax.ShapeDtypeStruct((NC, n, d), jnp.float32),
        mesh=_sc_mesh(),
        scratch_types=[
            pltpu.VMEM((CH // 2, K), jnp.int32),
            pltpu.VMEM((CH // 2, K), jnp.int32),
            pltpu.VMEM((K, d), jnp.float32),
            pltpu.VMEM((K, d), jnp.float32),
            pltpu.SemaphoreType.DMA,
            pltpu.SemaphoreType.DMA,
            pltpu.VMEM_SHARED((n, d), jnp.float32),
        ],
    )
    def scatter_kernel(h2_hbm, row_hbm, col_hbm, zeros_hbm, out_hbm,
                       colv, rowv, buf0, buf1, sem0, sem1, acc_sh):
        c = lax.axis_index("c")
        s = lax.axis_index("s")
        wid = s * NC + c
        sb, sz = _stripe(n, s)
        sh = CH // 2
        pltpu.sync_copy(zeros_hbm.at[pl.ds(sb, sz)], acc_sh.at[pl.ds(sb, sz)])
        plsc.subcore_barrier()

        for half in range(2):
            pltpu.sync_copy(col_hbm.at[wid, half], colv)
            pltpu.sync_copy(row_hbm.at[wid, half], rowv)

            pltpu.async_copy(h2_hbm.at[colv.at[0]], buf0, sem0)
            pltpu.async_copy(h2_hbm.at[colv.at[1]], buf1, sem1)

            def body(g, carry):
                j = 2 * g
                pltpu.make_async_copy(h2_hbm.at[colv.at[j]], buf0, sem0).wait()
                pltpu.sync_copy(buf0, acc_sh.at[rowv.at[j]], add=True)

                @pl.when(j + 2 < sh)
                def _():
                    pltpu.async_copy(h2_hbm.at[colv.at[j + 2]], buf0, sem0)

                pltpu.make_async_copy(h2_hbm.at[colv.at[j + 1]], buf1,
                                      sem1).wait()
                pltpu.sync_copy(buf1, acc_sh.at[rowv.at[j + 1]], add=True)

                @pl.when(j + 3 < sh)
                def _():
                    pltpu.async_copy(h2_hbm.at[colv.at[j + 3]], buf1, sem1)

                return carry

            lax.fori_loop(0, sh // 2, body, 0)

        plsc.subcore_barrier()
        pltpu.sync_copy(acc_sh.at[pl.ds(sb, sz)], out_hbm.at[c, pl.ds(sb, sz)])

    return scatter_kernel


def _h2_body(x_ref, wt_ref, b_ref, degp_ref, h2_ref):
    parts = degp_ref[...]
    deg = parts[0] + parts[1]
    dinv = lax.rsqrt(deg[:, 0:1] + 1.0)
    h = jnp.dot(x_ref[...], wt_ref[...], preferred_element_type=jnp.float32)
    h2_ref[...] = (h + b_ref[...]) * dinv


def _final_body(parts_ref, h2_ref, degp_ref, out_ref):
    dparts = degp_ref[...]
    deg = dparts[0] + dparts[1]
    dinv = lax.rsqrt(deg[:, 0:1] + 1.0)
    acc = parts_ref[0] + parts_ref[1] + h2_ref[...]
    out_ref[...] = acc * dinv


def kernel(x, edge_index, W, b):
    n, d_in = x.shape
    d_out = W.shape[0]
    e = edge_index.shape[1]
    assert e == NW * CH * K and n % NS == 0

    row = edge_index[0].astype(jnp.int32).reshape(NW, CH, K)
    col = edge_index[1].astype(jnp.int32).reshape(NW, CH, K)
    w_t = W.T
    b2 = b.reshape(1, d_out)
    ones_k = jnp.ones((K, DEGW), jnp.float32)
    zeros_nd = jnp.zeros((n, d_out), jnp.float32)

    degp = _make_deg_call(n, e)(row, ones_k, zeros_nd)

    blk = 1000
    grid = (n // blk,)
    h2 = pl.pallas_call(
        _h2_body,
        grid=grid,
        in_specs=[
            pl.BlockSpec((blk, d_in), lambda i: (i, 0)),
            pl.BlockSpec((d_in, d_out), lambda i: (0, 0)),
            pl.BlockSpec((1, d_out), lambda i: (0, 0)),
            pl.BlockSpec((NC, blk, DEGW), lambda i: (0, i, 0)),
        ],
        out_specs=pl.BlockSpec((blk, d_out), lambda i: (i, 0)),
        out_shape=jax.ShapeDtypeStruct((n, d_out), jnp.float32),
    )(x, w_t, b2, degp)

    row4 = row.reshape(NW, 2, CH // 2, K)
    col4 = col.reshape(NW, 2, CH // 2, K)
    parts = _make_scatter_call(n, d_out, e)(h2, row4, col4, zeros_nd)

    out = pl.pallas_call(
        _final_body,
        grid=grid,
        in_specs=[
            pl.BlockSpec((NC, blk, d_out), lambda i: (0, i, 0)),
            pl.BlockSpec((blk, d_out), lambda i: (i, 0)),
            pl.BlockSpec((NC, blk, DEGW), lambda i: (0, i, 0)),
        ],
        out_specs=pl.BlockSpec((blk, d_out), lambda i: (i, 0)),
        out_shape=jax.ShapeDtypeStruct((n, d_out), jnp.float32),
    )(parts, h2, degp)
    return out


# speedup vs baseline: 31.5281x; 1.0168x over previous
import functools

import jax
import jax.numpy as jnp
from jax import lax
from jax.experimental import pallas as pl
from jax.experimental.pallas import tpu as pltpu
from jax.experimental.pallas import tpu_sc as plsc

NC = 2
NS = 16
NW = NC * NS
K = 100
CH = 100
NQ = 4
QQ = CH // NQ
NB = 3
FB = 4
DEGW = 128


def _sc_mesh():
    return plsc.VectorSubcoreMesh(
        core_axis_name="c", subcore_axis_name="s", num_cores=NC, num_subcores=NS
    )


def _stripe(n, s):
    sz = ((n + NS - 1) // NS + 7) // 8 * 8
    base = jnp.minimum(s * sz, n - sz)
    return base, sz


def _make_deg_call(n, e):
    @functools.partial(
        pl.kernel,
        out_type=jax.ShapeDtypeStruct((NC, n, DEGW), jnp.float32),
        mesh=_sc_mesh(),
        scratch_types=[
            pltpu.VMEM((CH, K), jnp.int32),
            pltpu.VMEM((K, DEGW), jnp.float32),
            pltpu.VMEM_SHARED((n, DEGW), jnp.float32),
            pltpu.SemaphoreType.DMA,
        ],
    )
    def deg_kernel(row_hbm, ones_hbm, zeros_hbm, out_hbm, rowv, ones_v,
                   acc_sh, sem):
        c = lax.axis_index("c")
        s = lax.axis_index("s")
        wid = s * NC + c
        sb, sz = _stripe(n, s)
        pltpu.sync_copy(zeros_hbm.at[pl.ds(sb, sz)], acc_sh.at[pl.ds(sb, sz)])
        pltpu.sync_copy(ones_hbm, ones_v)
        pltpu.sync_copy(row_hbm.at[wid], rowv)
        plsc.subcore_barrier()

        def body(g, carry):
            j = g * FB
            for t in range(FB):
                pltpu.async_copy(ones_v, acc_sh.at[rowv.at[j + t]], sem,
                                 add=True)
            for t in range(FB):
                pltpu.make_async_copy(ones_v, acc_sh.at[rowv.at[j + t]],
                                      sem).wait()
            return carry

        lax.fori_loop(0, CH // FB, body, 0)
        plsc.subcore_barrier()
        pltpu.sync_copy(acc_sh.at[pl.ds(sb, sz)], out_hbm.at[c, pl.ds(sb, sz)])

    return deg_kernel


def _make_scatter_call(n, d, e):
    @functools.partial(
        pl.kernel,
        out_type=jax.ShapeDtypeStruct((NC, n, d), jnp.float32),
        mesh=_sc_mesh(),
        scratch_types=(
            [pltpu.VMEM((QQ, K), jnp.int32)] * 2
            + [pltpu.VMEM((K, d), jnp.float32)] * NB
            + [pltpu.SemaphoreType.DMA] * (2 * NB)
            + [pltpu.VMEM_SHARED((n, d), jnp.float32)]
        ),
    )
    def scatter_kernel(h2_hbm, row_hbm, col_hbm, zeros_hbm, out_hbm,
                       colv, rowv, *rest):
        bufs = rest[:NB]
        gsem = rest[NB:2 * NB]
        ssem = rest[2 * NB:3 * NB]
        acc_sh = rest[3 * NB]
        c = lax.axis_index("c")
        s = lax.axis_index("s")
        wid = s * NC + c
        sb, sz = _stripe(n, s)
        pltpu.sync_copy(zeros_hbm.at[pl.ds(sb, sz)], acc_sh.at[pl.ds(sb, sz)])
        plsc.subcore_barrier()

        for q in range(NQ):
            pltpu.sync_copy(col_hbm.at[wid, q], colv)
            pltpu.sync_copy(row_hbm.at[wid, q], rowv)

            pltpu.async_copy(h2_hbm.at[colv.at[0]], bufs[0], gsem[0])
            pltpu.async_copy(h2_hbm.at[colv.at[1]], bufs[1], gsem[1])

            def step(j, t):
                bb = (t + 2) % NB
                pltpu.make_async_copy(h2_hbm.at[colv.at[j]], bufs[t],
                                      gsem[t]).wait()
                pltpu.async_copy(bufs[t], acc_sh.at[rowv.at[j]], ssem[t],
                                 add=True)

                @pl.when(jnp.logical_and(j >= 1, j + 2 < QQ))
                def _():
                    pltpu.make_async_copy(bufs[bb], acc_sh.at[rowv.at[j]],
                                          ssem[bb]).wait()

                @pl.when(j + 2 < QQ)
                def _():
                    pltpu.async_copy(h2_hbm.at[colv.at[j + 2]], bufs[bb],
                                     gsem[bb])

            def body(g, carry):
                for t in range(NB):
                    step(NB * g + t, t)
                return carry

            lax.fori_loop(0, QQ // NB, body, 0)
            for j in range(QQ - QQ % NB, QQ):
                step(j, j % NB)
            for j in range(QQ - NB, QQ):
                pltpu.make_async_copy(bufs[j % NB], acc_sh.at[rowv.at[0]],
                                      ssem[j % NB]).wait()

        plsc.subcore_barrier()
        pltpu.sync_copy(acc_sh.at[pl.ds(sb, sz)], out_hbm.at[c, pl.ds(sb, sz)])

    return scatter_kernel


def _h2_body(x_ref, wt_ref, b_ref, degp_ref, h2_ref):
    parts = degp_ref[...]
    deg = parts[0] + parts[1]
    dinv = lax.rsqrt(deg[:, 0:1] + 1.0)
    h = jnp.dot(x_ref[...], wt_ref[...], preferred_element_type=jnp.float32)
    h2_ref[...] = (h + b_ref[...]) * dinv


def _final_body(parts_ref, h2_ref, degp_ref, out_ref):
    dparts = degp_ref[...]
    deg = dparts[0] + dparts[1]
    dinv = lax.rsqrt(deg[:, 0:1] + 1.0)
    acc = parts_ref[0] + parts_ref[1] + h2_ref[...]
    out_ref[...] = acc * dinv


def kernel(x, edge_index, W, b):
    n, d_in = x.shape
    d_out = W.shape[0]
    e = edge_index.shape[1]
    assert e == NW * CH * K and n % NS == 0

    row = edge_index[0].astype(jnp.int32)
    col = edge_index[1].astype(jnp.int32)
    row3 = row.reshape(NW, CH, K)
    w_t = W.T
    b2 = b.reshape(1, d_out)
    ones_k = jnp.ones((K, DEGW), jnp.float32)
    zeros_nd = jnp.zeros((n, d_out), jnp.float32)

    degp = _make_deg_call(n, e)(row3, ones_k, zeros_nd)

    blk = 1000
    grid = (n // blk,)
    h2 = pl.pallas_call(
        _h2_body,
        grid=grid,
        in_specs=[
            pl.BlockSpec((blk, d_in), lambda i: (i, 0)),
            pl.BlockSpec((d_in, d_out), lambda i: (0, 0)),
            pl.BlockSpec((1, d_out), lambda i: (0, 0)),
            pl.BlockSpec((NC, blk, DEGW), lambda i: (0, i, 0)),
        ],
        out_specs=pl.BlockSpec((blk, d_out), lambda i: (i, 0)),
        out_shape=jax.ShapeDtypeStruct((n, d_out), jnp.float32),
    )(x, w_t, b2, degp)

    row4 = row.reshape(NW, NQ, QQ, K)
    col4 = col.reshape(NW, NQ, QQ, K)
    parts = _make_scatter_call(n, d_out, e)(h2, row4, col4, zeros_nd)

    out = pl.pallas_call(
        _final_body,
        grid=grid,
        in_specs=[
            pl.BlockSpec((NC, blk, d_out), lambda i: (0, i, 0)),
            pl.BlockSpec((blk, d_out), lambda i: (i, 0)),
            pl.BlockSpec((NC, blk, DEGW), lambda i: (0, i, 0)),
        ],
        out_specs=pl.BlockSpec((blk, d_out), lambda i: (i, 0)),
        out_shape=jax.ShapeDtypeStruct((n, d_out), jnp.float32),
    )(parts, h2, degp)
    return out


# speedup vs baseline: 31.5634x; 1.0011x over previous
import functools

import jax
import jax.numpy as jnp
from jax import lax
from jax.experimental import pallas as pl
from jax.experimental.pallas import tpu as pltpu
from jax.experimental.pallas import tpu_sc as plsc

NC = 2
NS = 16
NW = NC * NS
K = 100
CH = 100
NQ = 4
QQ = CH // NQ
NB = 3
FB = 4
DEGW = 128


def _sc_mesh():
    return plsc.VectorSubcoreMesh(
        core_axis_name="c", subcore_axis_name="s", num_cores=NC, num_subcores=NS
    )


def _stripe(n, s):
    sz = ((n + NS - 1) // NS + 7) // 8 * 8
    base = jnp.minimum(s * sz, n - sz)
    return base, sz


def _make_deg_call(n, e):
    @functools.partial(
        pl.kernel,
        out_type=jax.ShapeDtypeStruct((NC, n, DEGW), jnp.float32),
        mesh=_sc_mesh(),
        scratch_types=(
            [pltpu.VMEM((CH, K), jnp.int32),
             pltpu.VMEM((K, DEGW), jnp.float32),
             pltpu.VMEM_SHARED((n, DEGW), jnp.float32)]
            + [pltpu.SemaphoreType.DMA] * FB
        ),
    )
    def deg_kernel(row_hbm, ones_hbm, zeros_hbm, out_hbm, rowv, ones_v,
                   acc_sh, *sems):
        c = lax.axis_index("c")
        s = lax.axis_index("s")
        wid = s * NC + c
        sb, sz = _stripe(n, s)
        pltpu.sync_copy(zeros_hbm.at[pl.ds(sb, sz)], acc_sh.at[pl.ds(sb, sz)])
        pltpu.sync_copy(ones_hbm, ones_v)
        pltpu.sync_copy(row_hbm.at[wid], rowv)
        plsc.subcore_barrier()

        def body(g, carry):
            for t in range(FB):
                j = g * FB + t

                @pl.when(j >= FB)
                def _():
                    pltpu.make_async_copy(ones_v, acc_sh.at[rowv.at[j]],
                                          sems[t]).wait()

                pltpu.async_copy(ones_v, acc_sh.at[rowv.at[j]], sems[t],
                                 add=True)
            return carry

        lax.fori_loop(0, CH // FB, body, 0)
        for t in range(FB):
            pltpu.make_async_copy(ones_v, acc_sh.at[rowv.at[0]],
                                  sems[t]).wait()
        plsc.subcore_barrier()
        pltpu.sync_copy(acc_sh.at[pl.ds(sb, sz)], out_hbm.at[c, pl.ds(sb, sz)])

    return deg_kernel


def _make_scatter_call(n, d, e):
    @functools.partial(
        pl.kernel,
        out_type=jax.ShapeDtypeStruct((NC, n, d), jnp.float32),
        mesh=_sc_mesh(),
        scratch_types=(
            [pltpu.VMEM((QQ, K), jnp.int32)] * 2
            + [pltpu.VMEM((K, d), jnp.float32)] * NB
            + [pltpu.SemaphoreType.DMA] * (2 * NB)
            + [pltpu.VMEM_SHARED((n, d), jnp.float32)]
        ),
    )
    def scatter_kernel(h2_hbm, row_hbm, col_hbm, zeros_hbm, out_hbm,
                       colv, rowv, *rest):
        bufs = rest[:NB]
        gsem = rest[NB:2 * NB]
        ssem = rest[2 * NB:3 * NB]
        acc_sh = rest[3 * NB]
        c = lax.axis_index("c")
        s = lax.axis_index("s")
        wid = s * NC + c
        sb, sz = _stripe(n, s)
        pltpu.sync_copy(zeros_hbm.at[pl.ds(sb, sz)], acc_sh.at[pl.ds(sb, sz)])
        plsc.subcore_barrier()

        for q in range(NQ):
            pltpu.sync_copy(col_hbm.at[wid, q], colv)
            pltpu.sync_copy(row_hbm.at[wid, q], rowv)

            pltpu.async_copy(h2_hbm.at[colv.at[0]], bufs[0], gsem[0])
            pltpu.async_copy(h2_hbm.at[colv.at[1]], bufs[1], gsem[1])

            def step(j, t):
                bb = (t + 2) % NB
                pltpu.make_async_copy(h2_hbm.at[colv.at[j]], bufs[t],
                                      gsem[t]).wait()
                pltpu.async_copy(bufs[t], acc_sh.at[rowv.at[j]], ssem[t],
                                 add=True)

                @pl.when(jnp.logical_and(j >= 1, j + 2 < QQ))
                def _():
                    pltpu.make_async_copy(bufs[bb], acc_sh.at[rowv.at[j]],
                                          ssem[bb]).wait()

                @pl.when(j + 2 < QQ)
                def _():
                    pltpu.async_copy(h2_hbm.at[colv.at[j + 2]], bufs[bb],
                                     gsem[bb])

            def body(g, carry):
                for t in range(NB):
                    step(NB * g + t, t)
                return carry

            lax.fori_loop(0, QQ // NB, body, 0)
            for j in range(QQ - QQ % NB, QQ):
                step(j, j % NB)
            for j in range(QQ - NB, QQ):
                pltpu.make_async_copy(bufs[j % NB], acc_sh.at[rowv.at[0]],
                                      ssem[j % NB]).wait()

        plsc.subcore_barrier()
        pltpu.sync_copy(acc_sh.at[pl.ds(sb, sz)], out_hbm.at[c, pl.ds(sb, sz)])

    return scatter_kernel


def _h2_body(x_ref, wt_ref, b_ref, degp_ref, h2_ref):
    parts = degp_ref[...]
    deg = parts[0] + parts[1]
    dinv = lax.rsqrt(deg[:, 0:1] + 1.0)
    h = jnp.dot(x_ref[...], wt_ref[...], preferred_element_type=jnp.float32)
    h2_ref[...] = (h + b_ref[...]) * dinv


def _final_body(parts_ref, h2_ref, degp_ref, out_ref):
    dparts = degp_ref[...]
    deg = dparts[0] + dparts[1]
    dinv = lax.rsqrt(deg[:, 0:1] + 1.0)
    acc = parts_ref[0] + parts_ref[1] + h2_ref[...]
    out_ref[...] = acc * dinv


def kernel(x, edge_index, W, b):
    n, d_in = x.shape
    d_out = W.shape[0]
    e = edge_index.shape[1]
    assert e == NW * CH * K and n % NS == 0

    row = edge_index[0].astype(jnp.int32)
    col = edge_index[1].astype(jnp.int32)
    row3 = row.reshape(NW, CH, K)
    w_t = W.T
    b2 = b.reshape(1, d_out)
    ones_k = jnp.ones((K, DEGW), jnp.float32)
    zeros_nd = jnp.zeros((n, d_out), jnp.float32)

    degp = _make_deg_call(n, e)(row3, ones_k, zeros_nd)

    blk = 1000
    grid = (n // blk,)
    h2 = pl.pallas_call(
        _h2_body,
        grid=grid,
        in_specs=[
            pl.BlockSpec((blk, d_in), lambda i: (i, 0)),
            pl.BlockSpec((d_in, d_out), lambda i: (0, 0)),
            pl.BlockSpec((1, d_out), lambda i: (0, 0)),
            pl.BlockSpec((NC, blk, DEGW), lambda i: (0, i, 0)),
        ],
        out_specs=pl.BlockSpec((blk, d_out), lambda i: (i, 0)),
        out_shape=jax.ShapeDtypeStruct((n, d_out), jnp.float32),
    )(x, w_t, b2, degp)

    row4 = row.reshape(NW, NQ, QQ, K)
    col4 = col.reshape(NW, NQ, QQ, K)
    parts = _make_scatter_call(n, d_out, e)(h2, row4, col4, zeros_nd)

    out = pl.pallas_call(
        _final_body,
        grid=grid,
        in_specs=[
            pl.BlockSpec((NC, blk, d_out), lambda i: (0, i, 0)),
            pl.BlockSpec((blk, d_out), lambda i: (i, 0)),
            pl.BlockSpec((NC, blk, DEGW), lambda i: (0, i, 0)),
        ],
        out_specs=pl.BlockSpec((blk, d_out), lambda i: (i, 0)),
        out_shape=jax.ShapeDtypeStruct((n, d_out), jnp.float32),
    )(parts, h2, degp)
    return out


# speedup vs baseline: 31.6788x; 1.0037x over previous
import functools

import jax
import jax.numpy as jnp
from jax import lax
from jax.experimental import pallas as pl
from jax.experimental.pallas import tpu as pltpu
from jax.experimental.pallas import tpu_sc as plsc

NC = 2
NS = 16
NW = NC * NS
K = 100
CH = 100
NQ = 4
QQ = CH // NQ
NB = 3
FB = 4
DEGW = 128


def _sc_mesh():
    return plsc.VectorSubcoreMesh(
        core_axis_name="c", subcore_axis_name="s", num_cores=NC, num_subcores=NS
    )


def _stripe(n, s):
    sz = ((n + NS - 1) // NS + 7) // 8 * 8
    base = jnp.minimum(s * sz, n - sz)
    return base, sz


def _make_deg_call(n, e):
    @functools.partial(
        pl.kernel,
        out_type=jax.ShapeDtypeStruct((NC, n, DEGW), jnp.float32),
        mesh=_sc_mesh(),
        scratch_types=(
            [pltpu.VMEM((CH, K), jnp.int32),
             pltpu.VMEM((K, DEGW), jnp.float32),
             pltpu.VMEM_SHARED((n, DEGW), jnp.float32)]
            + [pltpu.SemaphoreType.DMA] * FB
        ),
    )
    def deg_kernel(row_hbm, ones_hbm, zeros_hbm, out_hbm, rowv, ones_v,
                   acc_sh, *sems):
        c = lax.axis_index("c")
        s = lax.axis_index("s")
        wid = s * NC + c
        sb, sz = _stripe(n, s)
        pltpu.sync_copy(zeros_hbm.at[pl.ds(sb, sz)], acc_sh.at[pl.ds(sb, sz)])
        pltpu.sync_copy(ones_hbm, ones_v)
        pltpu.sync_copy(row_hbm.at[wid], rowv)
        plsc.subcore_barrier()

        def body(g, carry):
            for t in range(FB):
                j = g * FB + t

                @pl.when(j >= FB)
                def _():
                    pltpu.make_async_copy(ones_v, acc_sh.at[rowv.at[j]],
                                          sems[t]).wait()

                pltpu.async_copy(ones_v, acc_sh.at[rowv.at[j]], sems[t],
                                 add=True)
            return carry

        lax.fori_loop(0, CH // FB, body, 0)
        for t in range(FB):
            pltpu.make_async_copy(ones_v, acc_sh.at[rowv.at[0]],
                                  sems[t]).wait()
        plsc.subcore_barrier()
        pltpu.sync_copy(acc_sh.at[pl.ds(sb, sz)], out_hbm.at[c, pl.ds(sb, sz)])

    return deg_kernel


def _make_scatter_call(n, d, e):
    @functools.partial(
        pl.kernel,
        out_type=jax.ShapeDtypeStruct((NC, n, d), jnp.float32),
        mesh=_sc_mesh(),
        scratch_types=(
            [pltpu.VMEM((QQ, K), jnp.int32)] * 2
            + [pltpu.VMEM((K, d), jnp.float32)] * NB
            + [pltpu.SemaphoreType.DMA] * (2 * NB)
            + [pltpu.VMEM_SHARED((n, d), jnp.float32)]
        ),
    )
    def scatter_kernel(h2_hbm, row_hbm, col_hbm, zeros_hbm, out_hbm,
                       colv, rowv, *rest):
        bufs = rest[:NB]
        gsem = rest[NB:2 * NB]
        ssem = rest[2 * NB:3 * NB]
        acc_sh = rest[3 * NB]
        c = lax.axis_index("c")
        s = lax.axis_index("s")
        wid = s * NC + c
        sb, sz = _stripe(n, s)

        @pl.when(c == 0)
        def _():
            pltpu.sync_copy(h2_hbm.at[pl.ds(sb, sz)], acc_sh.at[pl.ds(sb, sz)])

        @pl.when(c != 0)
        def _():
            pltpu.sync_copy(zeros_hbm.at[pl.ds(sb, sz)],
                            acc_sh.at[pl.ds(sb, sz)])

        plsc.subcore_barrier()

        for q in range(NQ):
            pltpu.sync_copy(col_hbm.at[wid, q], colv)
            pltpu.sync_copy(row_hbm.at[wid, q], rowv)

            pltpu.async_copy(h2_hbm.at[colv.at[0]], bufs[0], gsem[0])
            pltpu.async_copy(h2_hbm.at[colv.at[1]], bufs[1], gsem[1])

            def step(j, t):
                bb = (t + 2) % NB
                pltpu.make_async_copy(h2_hbm.at[colv.at[j]], bufs[t],
                                      gsem[t]).wait()
                pltpu.async_copy(bufs[t], acc_sh.at[rowv.at[j]], ssem[t],
                                 add=True)

                @pl.when(jnp.logical_and(j >= 1, j + 2 < QQ))
                def _():
                    pltpu.make_async_copy(bufs[bb], acc_sh.at[rowv.at[j]],
                                          ssem[bb]).wait()

                @pl.when(j + 2 < QQ)
                def _():
                    pltpu.async_copy(h2_hbm.at[colv.at[j + 2]], bufs[bb],
                                     gsem[bb])

            def body(g, carry):
                for t in range(NB):
                    step(NB * g + t, t)
                return carry

            lax.fori_loop(0, QQ // NB, body, 0)
            for j in range(QQ - QQ % NB, QQ):
                step(j, j % NB)
            for j in range(QQ - NB, QQ):
                pltpu.make_async_copy(bufs[j % NB], acc_sh.at[rowv.at[0]],
                                      ssem[j % NB]).wait()

        plsc.subcore_barrier()
        pltpu.sync_copy(acc_sh.at[pl.ds(sb, sz)], out_hbm.at[c, pl.ds(sb, sz)])

    return scatter_kernel


def _h2_body(x_ref, wt_ref, b_ref, degp_ref, h2_ref):
    parts = degp_ref[...]
    deg = parts[0] + parts[1]
    dinv = lax.rsqrt(deg[:, 0:1] + 1.0)
    h = jnp.dot(x_ref[...], wt_ref[...], preferred_element_type=jnp.float32)
    h2_ref[...] = (h + b_ref[...]) * dinv


def _final_body(parts_ref, degp_ref, out_ref):
    dparts = degp_ref[...]
    deg = dparts[0] + dparts[1]
    dinv = lax.rsqrt(deg[:, 0:1] + 1.0)
    acc = parts_ref[0] + parts_ref[1]
    out_ref[...] = acc * dinv


def kernel(x, edge_index, W, b):
    n, d_in = x.shape
    d_out = W.shape[0]
    e = edge_index.shape[1]
    assert e == NW * CH * K and n % NS == 0

    row = edge_index[0].astype(jnp.int32)
    col = edge_index[1].astype(jnp.int32)
    row3 = row.reshape(NW, CH, K)
    w_t = W.T
    b2 = b.reshape(1, d_out)
    ones_k = jnp.ones((K, DEGW), jnp.float32)
    zeros_nd = jnp.zeros((n, d_out), jnp.float32)

    degp = _make_deg_call(n, e)(row3, ones_k, zeros_nd)

    blk = 1000
    grid = (n // blk,)
    h2 = pl.pallas_call(
        _h2_body,
        grid=grid,
        in_specs=[
            pl.BlockSpec((blk, d_in), lambda i: (i, 0)),
            pl.BlockSpec((d_in, d_out), lambda i: (0, 0)),
            pl.BlockSpec((1, d_out), lambda i: (0, 0)),
            pl.BlockSpec((NC, blk, DEGW), lambda i: (0, i, 0)),
        ],
        out_specs=pl.BlockSpec((blk, d_out), lambda i: (i, 0)),
        out_shape=jax.ShapeDtypeStruct((n, d_out), jnp.float32),
    )(x, w_t, b2, degp)

    row4 = row.reshape(NW, NQ, QQ, K)
    col4 = col.reshape(NW, NQ, QQ, K)
    parts = _make_scatter_call(n, d_out, e)(h2, row4, col4, zeros_nd)

    out = pl.pallas_call(
        _final_body,
        grid=grid,
        in_specs=[
            pl.BlockSpec((NC, blk, d_out), lambda i: (0, i, 0)),
            pl.BlockSpec((NC, blk, DEGW), lambda i: (0, i, 0)),
        ],
        out_specs=pl.BlockSpec((blk, d_out), lambda i: (i, 0)),
        out_shape=jax.ShapeDtypeStruct((n, d_out), jnp.float32),
    )(parts, degp)
    return out
